# baseline (speedup 1.0000x reference)
import jax
import jax.numpy as jnp
from jax import lax
from jax.experimental import pallas as pl
from jax.experimental.pallas import tpu as pltpu
from jax.experimental.pallas import tpu_sc as plsc

B, S, D = 4, 4096, 2048

_INFO = plsc.get_sparse_core_info()
NC, NS, L = _INFO.num_cores, _INFO.num_subcores, _INFO.num_lanes
NW = NC * NS
SPW = S // NW
R = 4
NCHUNK = SPW // R


def _body(x_hbm, tbl_hbm, out_hbm,
          tb0, tb1, xb0, xb1,
          semt0, semt1, semx0, semx1, semo0, semo1):
    wid = lax.axis_index("s") * NC + lax.axis_index("c")
    s_base = wid * SPW

    tbufs = (tb0, tb1)
    xbufs = (xb0, xb1)
    semts = (semt0, semt1)
    semxs = (semx0, semx1)
    semos = (semo0, semo1)

    def tbl_copy(c, p):
        s0 = s_base + c * R
        return pltpu.make_async_copy(
            tbl_hbm.at[pl.ds(s0, R)], tbufs[p], semts[p])

    def x_copy(c, p, b):
        s0 = s_base + c * R
        return pltpu.make_async_copy(
            x_hbm.at[pl.ds(b * S + s0, R)], xbufs[p].at[b], semxs[p])

    def out_copy(c, p, b):
        s0 = s_base + c * R
        return pltpu.make_async_copy(
            xbufs[p].at[b], out_hbm.at[pl.ds(b * S + s0, R)], semos[p])

    tbl_copy(0, 0).start()
    for b in range(B):
        x_copy(0, 0, b).start()

    def step(cc, carry):
        for q in range(2):
            c = cc * 2 + q
            p = q

            @pl.when(c + 1 < NCHUNK)
            def _():
                tbl_copy(c + 1, p ^ 1).start()
                @pl.when(c >= 1)
                def _():
                    for b in range(B):
                        out_copy(c - 1, p ^ 1, b).wait()
                for b in range(B):
                    x_copy(c + 1, p ^ 1, b).start()

            tbl_copy(c, p).wait()
            for b in range(B):
                x_copy(c, p, b).wait()

            tb = tbufs[p]
            xb = xbufs[p]

            def kloop(k, carry2):
                sl = pl.ds(k * L, L)
                for r in range(R):
                    t = tb[r, sl]
                    for b in range(B):
                        xb[b, r, sl] = xb[b, r, sl] + t
                return carry2

            lax.fori_loop(0, D // L, kloop, 0, unroll=2)

            for b in range(B):
                out_copy(c, p, b).start()
        return carry

    lax.fori_loop(0, NCHUNK // 2, step, 0)

    for b in range(B):
        out_copy(NCHUNK - 2, 0, b).wait()
    for b in range(B):
        out_copy(NCHUNK - 1, 1, b).wait()


@jax.jit
def kernel(encoded_tokens, position_table):
    x = encoded_tokens.reshape(B * S, D)
    run = pl.kernel(
        _body,
        out_type=jax.ShapeDtypeStruct((B * S, D), jnp.float32),
        mesh=plsc.VectorSubcoreMesh(core_axis_name="c", subcore_axis_name="s"),
        scratch_types=[
            pltpu.VMEM((R, D), jnp.float32),
            pltpu.VMEM((R, D), jnp.float32),
            pltpu.VMEM((B, R, D), jnp.float32),
            pltpu.VMEM((B, R, D), jnp.float32),
            pltpu.SemaphoreType.DMA,
            pltpu.SemaphoreType.DMA,
            pltpu.SemaphoreType.DMA,
            pltpu.SemaphoreType.DMA,
            pltpu.SemaphoreType.DMA,
            pltpu.SemaphoreType.DMA,
        ],
    )
    out = run(x, position_table)
    return out.reshape(B, S, D)
